# Initial kernel scaffold; baseline (speedup 1.0000x reference)
import jax
import jax.numpy as jnp
from jax.experimental import pallas as pl


def kernel(x, A):
    raise NotImplementedError("write your pallas kernel here")


# speedup vs baseline: 87.2183x; 87.2183x over previous
import jax
import jax.numpy as jnp
from jax.experimental import pallas as pl
from jax.experimental.pallas import tpu as pltpu

_DEPTH = 10
_NB_SPLIT = 2 ** _DEPTH - 1
_NB_NODES = 2 ** (_DEPTH + 1) - 1
_TILE = 128


def _latent_dt_body(x_ref, b_ref, e_ref, out_ref, xb_ref):
    xb_ref[...] = jax.lax.dot_general(
        x_ref[...], b_ref[...], (((1,), (1,)), ((), ())),
        preferred_element_type=jnp.float32)

    t = x_ref.shape[0]
    out_ref[:, 0:1] = jnp.ones((t, 1), jnp.float32)
    parent = jnp.ones((t, 1), jnp.float32)
    for d in range(_DEPTH):
        n = 1 << d
        spread = jax.lax.dot_general(
            parent, e_ref[:n, :2 * n], (((1,), (0,)), ((), ())),
            preferred_element_type=jnp.float32)
        child = jnp.clip(
            jnp.minimum(spread, xb_ref[:, 2 * n:4 * n]), 0.0, 1.0)
        out_ref[:, 2 * n - 1:4 * n - 1] = child
        parent = child


def _edge_matrix(A):
    signed = jnp.stack([A, -A], axis=1).reshape(2 * _NB_SPLIT, A.shape[1])
    return jnp.concatenate(
        [jnp.zeros((2, A.shape[1]), A.dtype), signed], axis=0)


def kernel(x, A):
    batch = x.shape[0]
    b_mat = _edge_matrix(A)
    cols = jnp.arange(2048, dtype=jnp.int32)
    rows = jnp.arange(1024, dtype=jnp.int32)
    e_mat = ((cols[None, :] >> 1) == rows[:, None]).astype(jnp.float32)
    return pl.pallas_call(
        _latent_dt_body,
        grid=(batch // _TILE,),
        in_specs=[
            pl.BlockSpec((_TILE, x.shape[1]), lambda i: (i, 0)),
            pl.BlockSpec((b_mat.shape[0], b_mat.shape[1]), lambda i: (0, 0)),
            pl.BlockSpec((1024, 2048), lambda i: (0, 0)),
        ],
        out_specs=pl.BlockSpec((_TILE, _NB_NODES), lambda i: (i, 0)),
        out_shape=jax.ShapeDtypeStruct((batch, _NB_NODES), jnp.float32),
        scratch_shapes=[pltpu.VMEM((_TILE, 2048), jnp.float32)],
        compiler_params=pltpu.CompilerParams(
            dimension_semantics=("arbitrary",)),
    )(x, b_mat, e_mat)


# speedup vs baseline: 140.7267x; 1.6135x over previous
import jax
import jax.numpy as jnp
from jax.experimental import pallas as pl
from jax.experimental.pallas import tpu as pltpu

_DEPTH = 10
_NB_SPLIT = 2 ** _DEPTH - 1
_NB_NODES = 2 ** (_DEPTH + 1) - 1
_TILE = 256


def _latent_dt_body(x_ref, b_ref, e_ref, out_ref, xb_ref):
    xb_ref[...] = jax.lax.dot_general(
        x_ref[...], b_ref[...], (((1,), (1,)), ((), ())),
        preferred_element_type=jnp.float32)

    t = x_ref.shape[0]
    out_ref[:, 0:1] = jnp.ones((t, 1), jnp.float32)
    parent = jnp.ones((t, 1), jnp.float32)
    for d in range(_DEPTH):
        n = 1 << d
        spread = jax.lax.dot_general(
            parent, e_ref[:n, :2 * n], (((1,), (0,)), ((), ())),
            preferred_element_type=jnp.float32)
        child = jnp.clip(
            jnp.minimum(spread, xb_ref[:, 2 * n:4 * n]), 0.0, 1.0)
        out_ref[:, 2 * n - 1:4 * n - 1] = child
        parent = child


def _edge_matrix(A):
    signed = jnp.stack([A, -A], axis=1).reshape(2 * _NB_SPLIT, A.shape[1])
    return jnp.concatenate(
        [jnp.zeros((2, A.shape[1]), A.dtype), signed], axis=0)


def kernel(x, A):
    batch = x.shape[0]
    b_mat = _edge_matrix(A)
    cols = jnp.arange(2048, dtype=jnp.int32)
    rows = jnp.arange(1024, dtype=jnp.int32)
    e_mat = ((cols[None, :] >> 1) == rows[:, None]).astype(jnp.float32)
    return pl.pallas_call(
        _latent_dt_body,
        grid=(batch // _TILE,),
        in_specs=[
            pl.BlockSpec((_TILE, x.shape[1]), lambda i: (i, 0)),
            pl.BlockSpec((b_mat.shape[0], b_mat.shape[1]), lambda i: (0, 0)),
            pl.BlockSpec((1024, 2048), lambda i: (0, 0)),
        ],
        out_specs=pl.BlockSpec((_TILE, _NB_NODES), lambda i: (i, 0)),
        out_shape=jax.ShapeDtypeStruct((batch, _NB_NODES), jnp.float32),
        scratch_shapes=[pltpu.VMEM((_TILE, 2048), jnp.float32)],
        compiler_params=pltpu.CompilerParams(
            dimension_semantics=("arbitrary",)),
    )(x, b_mat, e_mat)


# speedup vs baseline: 200.4478x; 1.4244x over previous
import jax
import jax.numpy as jnp
from jax.experimental import pallas as pl
from jax.experimental.pallas import tpu as pltpu

_DEPTH = 10
_NB_SPLIT = 2 ** _DEPTH - 1
_NB_NODES = 2 ** (_DEPTH + 1) - 1
_TILE = 512


def _latent_dt_body(x_ref, b_ref, e_ref, out_ref, xb_ref):
    xb_ref[...] = jax.lax.dot_general(
        x_ref[...], b_ref[...], (((1,), (1,)), ((), ())),
        preferred_element_type=jnp.float32)

    t = x_ref.shape[0]
    out_ref[:, 0:1] = jnp.ones((t, 1), jnp.float32)
    parent = jnp.ones((t, 1), jnp.float32)
    for d in range(_DEPTH):
        n = 1 << d
        spread = jax.lax.dot_general(
            parent, e_ref[:n, :2 * n], (((1,), (0,)), ((), ())),
            preferred_element_type=jnp.float32)
        child = jnp.clip(
            jnp.minimum(spread, xb_ref[:, 2 * n:4 * n]), 0.0, 1.0)
        out_ref[:, 2 * n - 1:4 * n - 1] = child
        parent = child


def _edge_matrix(A):
    signed = jnp.stack([A, -A], axis=1).reshape(2 * _NB_SPLIT, A.shape[1])
    return jnp.concatenate(
        [jnp.zeros((2, A.shape[1]), A.dtype), signed], axis=0)


def kernel(x, A):
    batch = x.shape[0]
    b_mat = _edge_matrix(A)
    cols = jnp.arange(2048, dtype=jnp.int32)
    rows = jnp.arange(1024, dtype=jnp.int32)
    e_mat = ((cols[None, :] >> 1) == rows[:, None]).astype(jnp.float32)
    return pl.pallas_call(
        _latent_dt_body,
        grid=(batch // _TILE,),
        in_specs=[
            pl.BlockSpec((_TILE, x.shape[1]), lambda i: (i, 0)),
            pl.BlockSpec((b_mat.shape[0], b_mat.shape[1]), lambda i: (0, 0)),
            pl.BlockSpec((1024, 2048), lambda i: (0, 0)),
        ],
        out_specs=pl.BlockSpec((_TILE, _NB_NODES), lambda i: (i, 0)),
        out_shape=jax.ShapeDtypeStruct((batch, _NB_NODES), jnp.float32),
        scratch_shapes=[pltpu.VMEM((_TILE, 2048), jnp.float32)],
        compiler_params=pltpu.CompilerParams(
            dimension_semantics=("arbitrary",)),
    )(x, b_mat, e_mat)


# speedup vs baseline: 237.2840x; 1.1838x over previous
import jax
import jax.numpy as jnp
from jax.experimental import pallas as pl
from jax.experimental.pallas import tpu as pltpu

_DEPTH = 10
_NB_SPLIT = 2 ** _DEPTH - 1
_NB_NODES = 2 ** (_DEPTH + 1) - 1
_TILE = 1024


def _latent_dt_body(x_ref, b_ref, e_ref, out_ref, xb_ref):
    xb_ref[...] = jax.lax.dot_general(
        x_ref[...], b_ref[...], (((1,), (1,)), ((), ())),
        preferred_element_type=jnp.float32)

    t = x_ref.shape[0]
    out_ref[:, 0:1] = jnp.ones((t, 1), jnp.float32)
    parent = jnp.ones((t, 1), jnp.float32)
    for d in range(_DEPTH):
        n = 1 << d
        spread = jax.lax.dot_general(
            parent, e_ref[:n, :2 * n], (((1,), (0,)), ((), ())),
            preferred_element_type=jnp.float32)
        child = jnp.clip(
            jnp.minimum(spread, xb_ref[:, 2 * n:4 * n]), 0.0, 1.0)
        out_ref[:, 2 * n - 1:4 * n - 1] = child
        parent = child


def _edge_matrix(A):
    signed = jnp.stack([A, -A], axis=1).reshape(2 * _NB_SPLIT, A.shape[1])
    return jnp.concatenate(
        [jnp.zeros((2, A.shape[1]), A.dtype), signed], axis=0)


def kernel(x, A):
    batch = x.shape[0]
    b_mat = _edge_matrix(A)
    cols = jnp.arange(2048, dtype=jnp.int32)
    rows = jnp.arange(1024, dtype=jnp.int32)
    e_mat = ((cols[None, :] >> 1) == rows[:, None]).astype(jnp.float32)
    return pl.pallas_call(
        _latent_dt_body,
        grid=(batch // _TILE,),
        in_specs=[
            pl.BlockSpec((_TILE, x.shape[1]), lambda i: (i, 0)),
            pl.BlockSpec((b_mat.shape[0], b_mat.shape[1]), lambda i: (0, 0)),
            pl.BlockSpec((1024, 2048), lambda i: (0, 0)),
        ],
        out_specs=pl.BlockSpec((_TILE, _NB_NODES), lambda i: (i, 0)),
        out_shape=jax.ShapeDtypeStruct((batch, _NB_NODES), jnp.float32),
        scratch_shapes=[pltpu.VMEM((_TILE, 2048), jnp.float32)],
        compiler_params=pltpu.CompilerParams(
            dimension_semantics=("arbitrary",)),
    )(x, b_mat, e_mat)


# speedup vs baseline: 242.2483x; 1.0209x over previous
import jax
import jax.numpy as jnp
from jax.experimental import pallas as pl
from jax.experimental.pallas import tpu as pltpu

_DEPTH = 10
_NB_SPLIT = 2 ** _DEPTH - 1
_NB_NODES = 2 ** (_DEPTH + 1) - 1
_TILE = 2048


def _latent_dt_body(x_ref, b_ref, e_ref, out_ref, xb_ref):
    xb_ref[...] = jax.lax.dot_general(
        x_ref[...], b_ref[...], (((1,), (1,)), ((), ())),
        preferred_element_type=jnp.float32)

    t = x_ref.shape[0]
    out_ref[:, 0:1] = jnp.ones((t, 1), jnp.float32)
    parent = jnp.ones((t, 1), jnp.float32)
    for d in range(_DEPTH):
        n = 1 << d
        spread = jax.lax.dot_general(
            parent, e_ref[:n, :2 * n], (((1,), (0,)), ((), ())),
            preferred_element_type=jnp.float32)
        child = jnp.clip(
            jnp.minimum(spread, xb_ref[:, 2 * n:4 * n]), 0.0, 1.0)
        out_ref[:, 2 * n - 1:4 * n - 1] = child
        parent = child


def _edge_matrix(A):
    signed = jnp.stack([A, -A], axis=1).reshape(2 * _NB_SPLIT, A.shape[1])
    return jnp.concatenate(
        [jnp.zeros((2, A.shape[1]), A.dtype), signed], axis=0)


def kernel(x, A):
    batch = x.shape[0]
    b_mat = _edge_matrix(A)
    cols = jnp.arange(1024, dtype=jnp.int32)
    rows = jnp.arange(512, dtype=jnp.int32)
    e_mat = ((cols[None, :] >> 1) == rows[:, None]).astype(jnp.float32)
    return pl.pallas_call(
        _latent_dt_body,
        grid=(batch // _TILE,),
        in_specs=[
            pl.BlockSpec((_TILE, x.shape[1]), lambda i: (i, 0)),
            pl.BlockSpec((b_mat.shape[0], b_mat.shape[1]), lambda i: (0, 0)),
            pl.BlockSpec((512, 1024), lambda i: (0, 0)),
        ],
        out_specs=pl.BlockSpec((_TILE, _NB_NODES), lambda i: (i, 0)),
        out_shape=jax.ShapeDtypeStruct((batch, _NB_NODES), jnp.float32),
        scratch_shapes=[pltpu.VMEM((_TILE, 2048), jnp.float32)],
        compiler_params=pltpu.CompilerParams(
            dimension_semantics=("arbitrary",)),
    )(x, b_mat, e_mat)


# speedup vs baseline: 270.3646x; 1.1161x over previous
import jax
import jax.numpy as jnp
from jax.experimental import pallas as pl
from jax.experimental.pallas import tpu as pltpu

_DEPTH = 10
_NB_SPLIT = 2 ** _DEPTH - 1
_NB_NODES = 2 ** (_DEPTH + 1) - 1
_TILE = 2048


def _latent_dt_body(x_ref, b_ref, e_ref, out_ref):
    x = x_ref[...]
    t = x.shape[0]
    out_ref[:, 0:1] = jnp.ones((t, 1), jnp.float32)
    parent = jnp.ones((t, 1), jnp.float32)
    for d in range(_DEPTH):
        n = 1 << d
        s = jax.lax.dot_general(
            x, b_ref[2 * n:4 * n, :], (((1,), (1,)), ((), ())),
            preferred_element_type=jnp.float32)
        spread = jax.lax.dot_general(
            parent, e_ref[:n, :2 * n], (((1,), (0,)), ((), ())),
            preferred_element_type=jnp.float32)
        child = jnp.clip(jnp.minimum(spread, s), 0.0, 1.0)
        out_ref[:, 2 * n - 1:4 * n - 1] = child
        parent = child


def _edge_matrix(A):
    signed = jnp.stack([A, -A], axis=1).reshape(2 * _NB_SPLIT, A.shape[1])
    return jnp.concatenate(
        [jnp.zeros((2, A.shape[1]), A.dtype), signed], axis=0)


def kernel(x, A):
    batch = x.shape[0]
    b_mat = _edge_matrix(A)
    cols = jnp.arange(1024, dtype=jnp.int32)
    rows = jnp.arange(512, dtype=jnp.int32)
    e_mat = ((cols[None, :] >> 1) == rows[:, None]).astype(jnp.float32)
    return pl.pallas_call(
        _latent_dt_body,
        grid=(batch // _TILE,),
        in_specs=[
            pl.BlockSpec((_TILE, x.shape[1]), lambda i: (i, 0)),
            pl.BlockSpec((b_mat.shape[0], b_mat.shape[1]), lambda i: (0, 0)),
            pl.BlockSpec((512, 1024), lambda i: (0, 0)),
        ],
        out_specs=pl.BlockSpec((_TILE, _NB_NODES), lambda i: (i, 0)),
        out_shape=jax.ShapeDtypeStruct((batch, _NB_NODES), jnp.float32),
        compiler_params=pltpu.CompilerParams(
            dimension_semantics=("arbitrary",)),
    )(x, b_mat, e_mat)
